# baseline (speedup 1.0000x reference)
import jax
import jax.numpy as jnp
from jax.experimental import pallas as pl
from jax.experimental.pallas import tpu as pltpu

_K = 10
_ROWS = 256


def _prep(x_ref, xs_ref, xstm2_ref, sq_ref):
    x = x_ref[...]
    c = jnp.mean(x, axis=0, keepdims=True)
    xc = x - c
    scale = 0.9 / jnp.max(jnp.abs(xc))
    xs = xc * scale
    xs_ref[...] = xs
    xst = xs.T
    xstm2_ref[...] = -2.0 * xst
    sq_ref[...] = jnp.sum(xst * xst, axis=0, keepdims=True)


def _extract_kth(w, k):
    m = None
    for i in range(k):
        m = jnp.min(w, axis=1, keepdims=True)
        if i < k - 1:
            w = jnp.where(w <= m, jnp.float32(jnp.inf), w)
    return m


def _affinity(c_ref, xs_ref, xstm2_ref, sq_ref, out_ref):
    xstm2 = xstm2_ref[...]
    xs_r = xs_ref[...]
    n = xstm2.shape[1]
    sq_full = sq_ref[...]
    sq_r = jnp.sum(xs_r * xs_r, axis=1, keepdims=True)
    dots = jax.lax.dot_general(
        xs_r, xstm2, (((1,), (0,)), ((), ())),
        preferred_element_type=jnp.float32)
    d = (sq_r + sq_full) + dots
    inf = jnp.float32(jnp.inf)
    a0 = jnp.full((d.shape[0], 128), inf, jnp.float32)
    a1 = a0
    a2 = a0
    a3 = a0
    for t in range(n // 128):
        v = d[:, t * 128:(t + 1) * 128]
        t0 = jnp.minimum(a0, v)
        v = jnp.maximum(a0, v)
        a0 = t0
        t1 = jnp.minimum(a1, v)
        v = jnp.maximum(a1, v)
        a1 = t1
        t2 = jnp.minimum(a2, v)
        v = jnp.maximum(a2, v)
        a2 = t2
        a3 = jnp.minimum(a3, v)
    kth_c = _extract_kth(jnp.concatenate([a0, a1, a2], axis=1), _K)
    hidden = jnp.min(a3, axis=1, keepdims=True) <= kth_c
    ok = jnp.logical_not(jnp.any(hidden))
    tau = jax.lax.cond(ok, lambda: kth_c, lambda: _extract_kth(d, _K))
    c1 = c_ref[0, 0]
    c0 = c_ref[0, 1]
    s = 1.0 / (1.0 + jnp.exp2(c1 * d + c0))
    out_ref[...] = jnp.where(d <= tau, s, 0.0)


def kernel(x, A, temperature, threshold):
    b, n, dim = x.shape
    x2 = x.reshape(n, dim)
    xs, xstm2, sq = pl.pallas_call(
        _prep,
        out_shape=(jax.ShapeDtypeStruct((n, dim), jnp.float32),
                   jax.ShapeDtypeStruct((dim, n), jnp.float32),
                   jax.ShapeDtypeStruct((1, n), jnp.float32)),
    )(x2)
    log2e = jnp.float32(1.4426950408889634)
    c1 = temperature * log2e
    c0 = -temperature * jnp.abs(threshold) * log2e
    consts = jnp.stack([c1, c0]).reshape(1, 2)
    out = pl.pallas_call(
        _affinity,
        grid=(n // _ROWS,),
        in_specs=[
            pl.BlockSpec(memory_space=pltpu.SMEM),
            pl.BlockSpec((_ROWS, dim), lambda i: (i, 0)),
            pl.BlockSpec((dim, n), lambda i: (0, 0)),
            pl.BlockSpec((1, n), lambda i: (0, 0)),
        ],
        out_specs=pl.BlockSpec((_ROWS, n), lambda i: (i, 0)),
        out_shape=jax.ShapeDtypeStruct((n, n), jnp.float32),
        compiler_params=pltpu.CompilerParams(
            dimension_semantics=("parallel",)),
    )(consts, xs, xstm2, sq)
    return (x, out.reshape(b, n, n))
